# Initial kernel scaffold; baseline (speedup 1.0000x reference)
import jax
import jax.numpy as jnp
from jax.experimental import pallas as pl


def kernel(x, beam_edge_index, column_edge_index, params):
    raise NotImplementedError("write your pallas kernel here")


# speedup vs baseline: 9.7630x; 9.7630x over previous
import jax
import jax.numpy as jnp
from jax import lax
from jax.experimental import pallas as pl
from jax.experimental.pallas import tpu as pltpu
from jax.experimental.pallas import tpu_sc as plsc

N = 50000
E = 400000
H = 64
HH = 32
EPS = 1e-5

NT = 16
CK = 96
CH = 264
EPAD = NT * CH * CK
NBAT = CH // 4
NPAD = 50176
SL = NPAD // NT
NDC = 33

_f32 = jnp.float32
_SDS = jax.ShapeDtypeStruct
_mesh = plsc.VectorSubcoreMesh(core_axis_name="c", subcore_axis_name="s")
_params = pltpu.CompilerParams(use_tc_tiling_on_sc=False)


def _dchunks():
    out = [(k * CK, CK) for k in range(32)]
    out.append((32 * CK, SL - 32 * CK))
    return out


def _deg_body(dstb, dstc, degb, degc, idx_d, ones_v, zb1, acc, isem, ssem, dsem):
    c = lax.axis_index("c")
    s = lax.axis_index("s")
    for k in range(CK // 16):
        ones_v[pl.ds(k * 16, 16)] = jnp.full((16,), 1.0, _f32)

    def zloop(i, carry):
        zb1[0, pl.ds(i * 16, 16)] = jnp.zeros((16,), _f32)
        zb1[1, pl.ds(i * 16, 16)] = jnp.zeros((16,), _f32)
        return carry

    lax.fori_loop(0, (SL // 2) // 16, zloop, 0)

    def side(dst3, out):
        for k in range(2):
            pltpu.async_copy(zb1.at[k],
                             acc.at[pl.ds(s * SL + k * (SL // 2), SL // 2)],
                             dsem)
        for k in range(2):
            pltpu.make_async_copy(zb1.at[k],
                                  acc.at[pl.ds(s * SL + k * (SL // 2), SL // 2)],
                                  dsem).wait()
        pltpu.sync_copy(dst3.at[s, pl.ds(0, 4)], idx_d.at[0])
        pltpu.async_copy(dst3.at[s, pl.ds(4, 4)], idx_d.at[1], isem)
        plsc.subcore_barrier()

        def sfire(slot, b):
            pltpu.async_copy(ones_v, acc.at[idx_d.at[slot, b]], ssem, add=True)

        def sdrain(slot, b):
            pltpu.make_async_copy(ones_v, acc.at[idx_d.at[slot, b]], ssem).wait()

        for b in range(4):
            sfire(0, b)

        def body(g, carry):
            p = lax.rem(g, 2)

            @pl.when(g + 1 < NBAT)
            def _():
                pltpu.make_async_copy(dst3.at[s, pl.ds((g + 1) * 4, 4)],
                                      idx_d.at[1 - p], isem).wait()
                for b in range(4):
                    sfire(1 - p, b)

            for b in range(4):
                sdrain(p, b)

            @pl.when(g + 2 < NBAT)
            def _():
                pltpu.async_copy(dst3.at[s, pl.ds((g + 2) * 4, 4)],
                                 idx_d.at[p], isem)
            return carry

        lax.fori_loop(0, NBAT, body, 0)
        plsc.subcore_barrier()
        for k in range(2):
            pltpu.async_copy(acc.at[pl.ds(s * SL + k * (SL // 2), SL // 2)],
                             zb1.at[k], dsem)
        for k in range(2):
            pltpu.make_async_copy(acc.at[pl.ds(s * SL + k * (SL // 2), SL // 2)],
                                  zb1.at[k], dsem).wait()
            pltpu.async_copy(zb1.at[k],
                             out.at[pl.ds(s * SL + k * (SL // 2), SL // 2)], dsem)
        for k in range(2):
            pltpu.make_async_copy(zb1.at[k],
                                  out.at[pl.ds(s * SL + k * (SL // 2), SL // 2)],
                                  dsem).wait()

    @pl.when(c == 0)
    def _():
        side(dstb, degb)

    @pl.when(c == 1)
    def _():
        side(dstc, degc)


_deg_call = pl.kernel(
    _deg_body,
    out_type=(_SDS((NPAD,), _f32), _SDS((NPAD,), _f32)),
    mesh=_mesh,
    compiler_params=_params,
    scratch_types=(
        pltpu.VMEM((2, 4, CK), jnp.int32),
        pltpu.VMEM((CK,), _f32),
        pltpu.VMEM((2, SL // 2), _f32),
        pltpu.VMEM_SHARED((NPAD,), _f32),
        pltpu.SemaphoreType.DMA,
        pltpu.SemaphoreType.DMA,
        pltpu.SemaphoreType.DMA,
    ),
)


def _conv_body(ybl, ybh, ycl, ych, srcb, dstb, srcc, dstc,
               obl, obh, ocl, och, idx_s, idx_d, rows, acc,
               isem, gsem, ssem, dsem, esem):
    c = lax.axis_index("c")
    s = lax.axis_index("s")

    def zero_rows():
        def zloop(i, carry):
            for b in range(8):
                rows[b, i, pl.ds(0, 16)] = jnp.zeros((16,), _f32)
                rows[b, i, pl.ds(16, 16)] = jnp.zeros((16,), _f32)
            return carry

        lax.fori_loop(0, CK, zloop, 0)

    def run(table, src3, dst3, out):
        zero_rows()
        ich = list(enumerate(_dchunks()))
        for grp in range(0, NDC, 8):
            sub = ich[grp:grp + 8]
            for k, (off, n) in sub:
                pltpu.async_copy(rows.at[k % 8, pl.ds(0, n)],
                                 acc.at[pl.ds(s * SL + off, n)], dsem)
            for k, (off, n) in sub:
                pltpu.make_async_copy(rows.at[k % 8, pl.ds(0, n)],
                                      acc.at[pl.ds(s * SL + off, n)], dsem).wait()
        pltpu.sync_copy(src3.at[s, pl.ds(0, 4)], idx_s.at[0])
        pltpu.sync_copy(dst3.at[s, pl.ds(0, 4)], idx_d.at[0])
        pltpu.async_copy(src3.at[s, pl.ds(4, 4)], idx_s.at[1], isem)
        pltpu.async_copy(dst3.at[s, pl.ds(4, 4)], idx_d.at[1], isem)
        plsc.subcore_barrier()

        def gfire(slot, b):
            pltpu.async_copy(table.at[idx_s.at[slot, b]],
                             rows.at[slot * 4 + b], gsem)

        def gdrain(slot, b):
            pltpu.make_async_copy(table.at[idx_s.at[slot, b]],
                                  rows.at[slot * 4 + b], gsem).wait()

        def sfire(slot, b):
            pltpu.async_copy(rows.at[slot * 4 + b],
                             acc.at[idx_d.at[slot, b]], ssem, add=True)

        def sdrain(slot, b):
            pltpu.make_async_copy(rows.at[slot * 4 + b],
                                  acc.at[idx_d.at[slot, b]], ssem).wait()

        for b in range(4):
            gfire(0, b)

        def body(g, carry):
            p = lax.rem(g, 2)
            for b in range(4):
                gdrain(p, b)
            for b in range(4):
                sfire(p, b)

            @pl.when(g + 1 < NBAT)
            def _():
                pltpu.make_async_copy(src3.at[s, pl.ds((g + 1) * 4, 4)],
                                      idx_s.at[1 - p], isem).wait()
                pltpu.make_async_copy(dst3.at[s, pl.ds((g + 1) * 4, 4)],
                                      idx_d.at[1 - p], isem).wait()
                for b in range(4):
                    gfire(1 - p, b)

            for b in range(4):
                sdrain(p, b)

            @pl.when(g + 2 < NBAT)
            def _():
                pltpu.async_copy(src3.at[s, pl.ds((g + 2) * 4, 4)],
                                 idx_s.at[p], isem)
                pltpu.async_copy(dst3.at[s, pl.ds((g + 2) * 4, 4)],
                                 idx_d.at[p], isem)
            return carry

        lax.fori_loop(0, NBAT, body, 0)
        plsc.subcore_barrier()

        ch = _dchunks()

        def h1(k):
            off, n = ch[k]
            pltpu.async_copy(acc.at[pl.ds(s * SL + off, n)],
                             rows.at[k % 8, pl.ds(0, n)], dsem)

        def h1w(k):
            off, n = ch[k]
            pltpu.make_async_copy(acc.at[pl.ds(s * SL + off, n)],
                                  rows.at[k % 8, pl.ds(0, n)], dsem).wait()

        def h2(k):
            off, n = ch[k]
            pltpu.async_copy(rows.at[k % 8, pl.ds(0, n)],
                             out.at[pl.ds(s * SL + off, n)], esem)

        def h2w(k):
            off, n = ch[k]
            pltpu.make_async_copy(rows.at[k % 8, pl.ds(0, n)],
                                  out.at[pl.ds(s * SL + off, n)], esem).wait()

        for k in range(4):
            h1(k)
        for k in range(NDC):
            h1w(k)
            h2(k)
            if k >= 4:
                h2w(k - 4)
            if k + 4 < NDC:
                h1(k + 4)
        for k in range(NDC - 4, NDC):
            h2w(k)
        plsc.subcore_barrier()

    @pl.when(c == 0)
    def _():
        run(ybl, srcb, dstb, obl)
        run(ycl, srcc, dstc, ocl)

    @pl.when(c == 1)
    def _():
        run(ybh, srcb, dstb, obh)
        run(ych, srcc, dstc, och)


_conv_call = pl.kernel(
    _conv_body,
    out_type=tuple(_SDS((NPAD, HH), _f32) for _ in range(4)),
    mesh=_mesh,
    compiler_params=_params,
    scratch_types=(
        pltpu.VMEM((2, 4, CK), jnp.int32),
        pltpu.VMEM((2, 4, CK), jnp.int32),
        pltpu.VMEM((8, CK, HH), _f32),
        pltpu.VMEM_SHARED((NPAD, HH), _f32),
        pltpu.SemaphoreType.DMA,
        pltpu.SemaphoreType.DMA,
        pltpu.SemaphoreType.DMA,
        pltpu.SemaphoreType.DMA,
        pltpu.SemaphoreType.DMA,
    ),
)


def _pair_body(abl, abh, bbl, bbh, acl, ach, bcl, bch,
               srcb, dstb, srcc, dstc,
               pbl, pbh, pcl, pch, idx_s, idx_d, rows,
               isem, asem, csem, wsem):
    c = lax.axis_index("c")
    s = lax.axis_index("s")
    base = s * (CH * CK)

    def run(ta, tb, src3, dst3, out):
        pltpu.sync_copy(src3.at[s, pl.ds(0, 4)], idx_s.at[0])
        pltpu.sync_copy(dst3.at[s, pl.ds(0, 4)], idx_d.at[0])
        pltpu.async_copy(src3.at[s, pl.ds(4, 4)], idx_s.at[1], isem)
        pltpu.async_copy(dst3.at[s, pl.ds(4, 4)], idx_d.at[1], isem)

        def g1fire(slot, b):
            pltpu.async_copy(ta.at[idx_s.at[slot, b]],
                             rows.at[slot * 4 + b], asem)

        def g1drain(slot, b):
            pltpu.make_async_copy(ta.at[idx_s.at[slot, b]],
                                  rows.at[slot * 4 + b], asem).wait()

        def g2fire(slot, b):
            pltpu.async_copy(tb.at[idx_d.at[slot, b]],
                             rows.at[slot * 4 + b], csem, add=True)

        def g2drain(slot, b):
            pltpu.make_async_copy(tb.at[idx_d.at[slot, b]],
                                  rows.at[slot * 4 + b], csem).wait()

        def wref(g, b):
            return out.at[pl.ds(base + (g * 4 + b) * CK, CK)]

        def wfire(g, slot, b):
            pltpu.async_copy(rows.at[slot * 4 + b], wref(g, b), wsem)

        def wdrain(g, slot, b):
            pltpu.make_async_copy(rows.at[slot * 4 + b], wref(g, b), wsem).wait()

        for b in range(4):
            g1fire(0, b)

        def body(g, carry):
            p = lax.rem(g, 2)
            for b in range(4):
                g1drain(p, b)
            for b in range(4):
                g2fire(p, b)

            @pl.when(g + 1 < NBAT)
            def _():
                pltpu.make_async_copy(src3.at[s, pl.ds((g + 1) * 4, 4)],
                                      idx_s.at[1 - p], isem).wait()
                pltpu.make_async_copy(dst3.at[s, pl.ds((g + 1) * 4, 4)],
                                      idx_d.at[1 - p], isem).wait()

            for b in range(4):
                g2drain(p, b)
            for b in range(4):
                wfire(g, p, b)

            @pl.when(g + 1 < NBAT)
            def _():
                for b in range(4):
                    g1fire(1 - p, b)

            for b in range(4):
                wdrain(g, p, b)

            @pl.when(g + 2 < NBAT)
            def _():
                pltpu.async_copy(src3.at[s, pl.ds((g + 2) * 4, 4)],
                                 idx_s.at[p], isem)
                pltpu.async_copy(dst3.at[s, pl.ds((g + 2) * 4, 4)],
                                 idx_d.at[p], isem)
            return carry

        lax.fori_loop(0, NBAT, body, 0)

    @pl.when(c == 0)
    def _():
        run(abl, bbl, srcb, dstb, pbl)
        run(acl, bcl, srcc, dstc, pcl)

    @pl.when(c == 1)
    def _():
        run(abh, bbh, srcb, dstb, pbh)
        run(ach, bch, srcc, dstc, pch)


_pair_call = pl.kernel(
    _pair_body,
    out_type=tuple(_SDS((EPAD, HH), _f32) for _ in range(4)),
    mesh=_mesh,
    compiler_params=_params,
    scratch_types=(
        pltpu.VMEM((2, 4, CK), jnp.int32),
        pltpu.VMEM((2, 4, CK), jnp.int32),
        pltpu.VMEM((8, CK, HH), _f32),
        pltpu.SemaphoreType.DMA,
        pltpu.SemaphoreType.DMA,
        pltpu.SemaphoreType.DMA,
        pltpu.SemaphoreType.DMA,
    ),
)


BM = 2000
_GN = N // BM
_GE = E // BM


def _tc1_body(x_ref, degb, degc, encW, encb, gam, bet, w1b, w1c,
              obl, obh, ocl, och):
    h = jnp.maximum(jnp.dot(x_ref[...], encW[...],
                            preferred_element_type=_f32) + encb[...], 0.0)
    h = h * gam[...] + bet[...]
    dinvb = lax.rsqrt(degb[...] + 1.0)
    dinvc = lax.rsqrt(degc[...] + 1.0)
    yb = jnp.dot(h, w1b[...], preferred_element_type=_f32) * dinvb
    yc = jnp.dot(h, w1c[...], preferred_element_type=_f32) * dinvc
    obl[...] = yb[:, :HH]
    obh[...] = yb[:, HH:]
    ocl[...] = yc[:, :HH]
    och[...] = yc[:, HH:]


def _tc1(xp, degb, degc, encW, encb, gam, bet, w1b, w1c):
    bs = pl.BlockSpec
    row = lambda i: (i, 0)
    full = lambda i: (0, 0)
    return pl.pallas_call(
        _tc1_body,
        grid=(_GN,),
        in_specs=[
            bs((BM, 8), row), bs((BM, 1), row), bs((BM, 1), row),
            bs((8, H), full), bs((1, H), full), bs((1, H), full),
            bs((1, H), full), bs((H, H), full), bs((H, H), full),
        ],
        out_specs=[bs((BM, HH), row)] * 4,
        out_shape=[_SDS((N, HH), _f32)] * 4,
    )(xp, degb, degc, encW, encb, gam, bet, w1b, w1c)


def _mid_body(abl, abh, acl, ach, ybl, ybh, ycl, ych, degb, degc,
              bb, bc, w2b, w2c, obl, obh, ocl, och, *, relu):
    dinvb = lax.rsqrt(degb[...] + 1.0)
    dinvc = lax.rsqrt(degc[...] + 1.0)
    accb = jnp.concatenate([abl[...], abh[...]], axis=1)
    accc = jnp.concatenate([acl[...], ach[...]], axis=1)
    yb = jnp.concatenate([ybl[...], ybh[...]], axis=1)
    yc = jnp.concatenate([ycl[...], ych[...]], axis=1)
    hn = dinvb * (accb + yb) + bb[...] + dinvc * (accc + yc) + bc[...]
    if relu:
        hn = jnp.maximum(hn, 0.0)
    zb = jnp.dot(hn, w2b[...], preferred_element_type=_f32) * dinvb
    zc = jnp.dot(hn, w2c[...], preferred_element_type=_f32) * dinvc
    obl[...] = zb[:, :HH]
    obh[...] = zb[:, HH:]
    ocl[...] = zc[:, :HH]
    och[...] = zc[:, HH:]


def _tc3(acc, y, degb, degc, bb, bc, w2b, w2c):
    bs = pl.BlockSpec
    row = lambda i: (i, 0)
    full = lambda i: (0, 0)
    return pl.pallas_call(
        lambda *refs: _mid_body(*refs, relu=True),
        grid=(_GN,),
        in_specs=[bs((BM, HH), row)] * 4 + [bs((BM, HH), row)] * 4 + [
            bs((BM, 1), row), bs((BM, 1), row),
            bs((1, H), full), bs((1, H), full),
            bs((H, H), full), bs((H, H), full),
        ],
        out_specs=[bs((BM, HH), row)] * 4,
        out_shape=[_SDS((N, HH), _f32)] * 4,
    )(*acc, *y, degb, degc, bb, bc, w2b, w2c)


def _tc5_body(abl, abh, acl, ach, ybl, ybh, ycl, ych, degb, degc,
              bb, bc, wcat, bcat, *outs):
    dinvb = lax.rsqrt(degb[...] + 1.0)
    dinvc = lax.rsqrt(degc[...] + 1.0)
    accb = jnp.concatenate([abl[...], abh[...]], axis=1)
    accc = jnp.concatenate([acl[...], ach[...]], axis=1)
    yb = jnp.concatenate([ybl[...], ybh[...]], axis=1)
    yc = jnp.concatenate([ycl[...], ych[...]], axis=1)
    h2 = dinvb * (accb + yb) + bb[...] + dinvc * (accc + yc) + bc[...]
    p = jnp.dot(h2, wcat[...], preferred_element_type=_f32) + bcat[...]
    for k in range(8):
        outs[k][...] = p[:, k * HH:(k + 1) * HH]


def _tc5(acc, y, degb, degc, bb, bc, wcat, bcat):
    bs = pl.BlockSpec
    row = lambda i: (i, 0)
    full = lambda i: (0, 0)
    return pl.pallas_call(
        _tc5_body,
        grid=(_GN,),
        in_specs=[bs((BM, HH), row)] * 4 + [bs((BM, HH), row)] * 4 + [
            bs((BM, 1), row), bs((BM, 1), row),
            bs((1, H), full), bs((1, H), full),
            bs((H, 4 * H), full), bs((1, 4 * H), full),
        ],
        out_specs=[bs((BM, HH), row)] * 8,
        out_shape=[_SDS((N, HH), _f32)] * 8,
    )(*acc, *y, degb, degc, bb, bc, wcat, bcat)


def _tc7_body(pbl, pbh, pcl, pch, w2b, b2b, w2c, b2c, outb, outc):
    hb = jnp.maximum(jnp.concatenate([pbl[...], pbh[...]], axis=1), 0.0)
    hc = jnp.maximum(jnp.concatenate([pcl[...], pch[...]], axis=1), 0.0)
    rb = jnp.dot(hb, w2b[...], preferred_element_type=_f32) + b2b[...]
    rc = jnp.dot(hc, w2c[...], preferred_element_type=_f32) + b2c[...]
    outb[...] = rb[:, :2]
    outc[...] = rc[:, :2]


def _tc7(pre, w2b, b2b, w2c, b2c):
    bs = pl.BlockSpec
    row = lambda i: (i, 0)
    full = lambda i: (0, 0)
    return pl.pallas_call(
        _tc7_body,
        grid=(_GE,),
        in_specs=[bs((BM, HH), row)] * 4 + [
            bs((H, 128), full), bs((1, 128), full),
            bs((H, 128), full), bs((1, 128), full),
        ],
        out_specs=[bs((BM, 2), row)] * 2,
        out_shape=[_SDS((E, 2), _f32)] * 2,
    )(*pre, w2b, b2b, w2c, b2c)


def kernel(x, beam_edge_index, column_edge_index, params):
    p = params
    i32 = jnp.int32
    npad = EPAD - E
    padg = (jnp.arange(npad, dtype=i32) * 8) % 4096
    pads = N + (jnp.arange(npad, dtype=i32) % (NPAD - N))

    def prep(idx, pad):
        return jnp.concatenate([idx, pad]).reshape(NT, CH, CK)

    srcb_g = prep(beam_edge_index[0], padg)
    dstb_g = prep(beam_edge_index[1], padg)
    dstb_s = prep(beam_edge_index[1], pads)
    srcc_g = prep(column_edge_index[0], padg)
    dstc_g = prep(column_edge_index[1], padg)
    dstc_s = prep(column_edge_index[1], pads)

    degb_f, degc_f = _deg_call(dstb_s, dstc_s)
    degb = degb_f[:N].reshape(N, 1)
    degc = degc_f[:N].reshape(N, 1)

    r = 1.0 / jnp.sqrt(jnp.float32(1.0 + EPS))
    xp = jnp.concatenate([x, jnp.zeros((N, 5), _f32)], axis=1)
    encW = jnp.concatenate([p['enc_W'], jnp.zeros((5, H), _f32)], axis=0)
    y1 = _tc1(xp, degb, degc, encW, p['enc_b'].reshape(1, H),
              (p['enc_gamma'] * r).reshape(1, H), p['enc_beta'].reshape(1, H),
              p['g1_beam_W'], p['g1_col_W'])
    acc1 = _conv_call(y1[0], y1[1], y1[2], y1[3],
                      srcb_g, dstb_s, srcc_g, dstc_s)
    y2 = _tc3(acc1, y1, degb, degc,
              p['g1_beam_b'].reshape(1, H), p['g1_col_b'].reshape(1, H),
              p['g2_beam_W'], p['g2_col_W'])
    acc2 = _conv_call(y2[0], y2[1], y2[2], y2[3],
                      srcb_g, dstb_s, srcc_g, dstc_s)
    wcat = jnp.concatenate([p['bp_W1'][:H], p['bp_W1'][H:],
                            p['cp_W1'][:H], p['cp_W1'][H:]], axis=1)
    zh = jnp.zeros((H,), _f32)
    bcat = jnp.concatenate([p['bp_b1'], zh, p['cp_b1'], zh]).reshape(1, 4 * H)
    ab = _tc5(acc2, y2, degb, degc,
              p['g2_beam_b'].reshape(1, H), p['g2_col_b'].reshape(1, H),
              wcat, bcat)
    pre = _pair_call(ab[0], ab[1], ab[2], ab[3], ab[4], ab[5], ab[6], ab[7],
                     srcb_g, dstb_g, srcc_g, dstc_g)
    w2b = jnp.zeros((H, 128), _f32).at[:, :2].set(
        (p['bp_gamma'] * r)[:, None] * p['bp_W2'])
    b2b = jnp.zeros((1, 128), _f32).at[0, :2].set(
        p['bp_beta'] @ p['bp_W2'] + p['bp_b2'])
    w2c = jnp.zeros((H, 128), _f32).at[:, :2].set(
        (p['cp_gamma'] * r)[:, None] * p['cp_W2'])
    b2c = jnp.zeros((1, 128), _f32).at[0, :2].set(
        p['cp_beta'] @ p['cp_W2'] + p['cp_b2'])
    beam_preds, column_preds = _tc7(pre, w2b, b2b, w2c, b2c)
    edge_types = jnp.concatenate([jnp.zeros((E,), i32), jnp.ones((E,), i32)])
    return (beam_preds, column_preds, edge_types)
